# baseline (speedup 1.0000x reference)
import functools

import jax
import jax.numpy as jnp
from jax import lax
from jax.experimental import pallas as pl
from jax.experimental.pallas import tpu as pltpu
from jax.experimental.pallas import tpu_sc as plsc

N_SPARSE = 10000
N_DENSE = 50000
E_SPARSE = 160000
E_DENSE = 800000
CLOSEST = 3
E_ML = N_DENSE * CLOSEST
LAT = 64
PASSES = 8

NW = 32
GROUP = 128
SLOTS = 8
DEPTH = 4
EDGE_QUANT = NW * GROUP * SLOTS

QC_SPARSE = 2560
QC_DENSE = 12544
NPAD_SPARSE = 4 * QC_SPARSE
NPAD_DENSE = 4 * QC_DENSE

DUMMY_RECV = 0xFFFF


def _pad_rows(x, n):
    return jnp.pad(x, ((0, n - x.shape[0]), (0, 0)))


def _pad_to(e):
    return ((e + EDGE_QUANT - 1) // EDGE_QUANT) * EDGE_QUANT


def _mlp1_body(x_ref, w1_ref, b1_ref, w2_ref, b2_ref, o_ref):
    h = jnp.dot(x_ref[...], w1_ref[...], preferred_element_type=jnp.float32)
    h = jnp.maximum(h + b1_ref[...], 0.0)
    o_ref[...] = jnp.dot(h, w2_ref[...], preferred_element_type=jnp.float32) + b2_ref[...]


def _mlp1(x, p, block=2048):
    n, din = x.shape
    dout = p["W2"].shape[1]
    return pl.pallas_call(
        _mlp1_body,
        grid=(pl.cdiv(n, block),),
        in_specs=[
            pl.BlockSpec((block, din), lambda i: (i, 0)),
            pl.BlockSpec((din, LAT), lambda i: (0, 0)),
            pl.BlockSpec((1, LAT), lambda i: (0, 0)),
            pl.BlockSpec((LAT, dout), lambda i: (0, 0)),
            pl.BlockSpec((1, dout), lambda i: (0, 0)),
        ],
        out_specs=pl.BlockSpec((block, dout), lambda i: (i, 0)),
        out_shape=jax.ShapeDtypeStruct((n, dout), jnp.float32),
    )(x, p["W1"], p["b1"].reshape(1, -1), p["W2"], p["b2"].reshape(1, -1))


def _mlp2_body(a_ref, b_ref, w1a_ref, w1b_ref, b1_ref, w2_ref, b2_ref, o_ref,
               *, residual):
    h = jnp.dot(a_ref[...], w1a_ref[...], preferred_element_type=jnp.float32)
    h = h + jnp.dot(b_ref[...], w1b_ref[...], preferred_element_type=jnp.float32)
    h = jnp.maximum(h + b1_ref[...], 0.0)
    o = jnp.dot(h, w2_ref[...], preferred_element_type=jnp.float32) + b2_ref[...]
    if residual:
        o = o + a_ref[...]
    o_ref[...] = o


def _mlp2(a, b, p, residual, block=2048):
    n = a.shape[0]
    return pl.pallas_call(
        functools.partial(_mlp2_body, residual=residual),
        grid=(pl.cdiv(n, block),),
        in_specs=[
            pl.BlockSpec((block, LAT), lambda i: (i, 0)),
            pl.BlockSpec((block, LAT), lambda i: (i, 0)),
            pl.BlockSpec((LAT, LAT), lambda i: (0, 0)),
            pl.BlockSpec((LAT, LAT), lambda i: (0, 0)),
            pl.BlockSpec((1, LAT), lambda i: (0, 0)),
            pl.BlockSpec((LAT, LAT), lambda i: (0, 0)),
            pl.BlockSpec((1, LAT), lambda i: (0, 0)),
        ],
        out_specs=pl.BlockSpec((block, LAT), lambda i: (i, 0)),
        out_shape=jax.ShapeDtypeStruct((n, LAT), jnp.float32),
    )(a, b, p["W1"][:LAT], p["W1"][LAT:], p["b1"].reshape(1, -1),
      p["W2"], p["b2"].reshape(1, -1))


def _sc_gather(table, idx):
    epad = idx.shape[0]
    k = epad // (NW * GROUP)
    idx3 = idx.reshape(NW, k, GROUP)
    mesh = plsc.VectorSubcoreMesh(core_axis_name="c", subcore_axis_name="s")

    @functools.partial(
        pl.kernel,
        mesh=mesh,
        out_type=jax.ShapeDtypeStruct((epad, LAT), jnp.float32),
        compiler_params=pltpu.CompilerParams(use_tc_tiling_on_sc=False),
        scratch_types=(
            [pltpu.VMEM((k, GROUP), jnp.int32)]
            + [pltpu.VMEM((GROUP, LAT), jnp.float32) for _ in range(DEPTH)]
            + [pltpu.SemaphoreType.DMA for _ in range(DEPTH)]
            + [pltpu.SemaphoreType.DMA]
        ),
    )
    def gk(table_hbm, idx_hbm, out_hbm, idx_v, *rest):
        bufs = rest[:DEPTH]
        gsems = rest[DEPTH:2 * DEPTH]
        wsem = rest[2 * DEPTH]
        wid = lax.axis_index("s") * 2 + lax.axis_index("c")
        pltpu.sync_copy(idx_hbm.at[wid], idx_v)
        base = wid * (k * GROUP)

        for b in range(DEPTH):
            pltpu.async_copy(table_hbm.at[idx_v.at[b]], bufs[b], gsems[b])

        def body(sb, carry):
            g = sb * DEPTH
            whs = []
            for b in range(DEPTH):
                j = g + b
                pltpu.make_async_copy(table_hbm.at[idx_v.at[j]], bufs[b],
                                      gsems[b]).wait()
                whs.append(pltpu.async_copy(
                    bufs[b], out_hbm.at[pl.ds(base + j * GROUP, GROUP)], wsem))
            for wh in whs:
                wh.wait()
            for b in range(DEPTH):
                nj = g + DEPTH + b

                @pl.when(nj < k)
                def _():
                    pltpu.async_copy(table_hbm.at[idx_v.at[nj]], bufs[b],
                                     gsems[b])
            return carry

        lax.fori_loop(0, k // DEPTH, body, 0)

    return gk(table, idx3)


def _sc_segsum(vals, idx, qc):
    epad = vals.shape[0]
    k2 = epad // (16 * GROUP)
    r2 = idx.reshape(-1, 2, 16)
    packed = r2[:, 0, :] | (r2[:, 1, :] << 16)
    idx3 = packed.reshape(16, k2, 64)
    rows_per_tile = qc // 16
    zblocks = rows_per_tile // 16
    mesh = plsc.VectorSubcoreMesh(core_axis_name="c", subcore_axis_name="s")

    @functools.partial(
        pl.kernel,
        mesh=mesh,
        out_type=jax.ShapeDtypeStruct((4 * qc, LAT), jnp.float32),
        compiler_params=pltpu.CompilerParams(use_tc_tiling_on_sc=False),
        scratch_types=(
            [pltpu.VMEM((GROUP, LAT), jnp.float32) for _ in range(DEPTH)]
            + [pltpu.SemaphoreType.DMA for _ in range(DEPTH + 2)]
            + [pltpu.VMEM((DEPTH, 64), jnp.int32) for _ in range(2)]
            + [
                pltpu.VMEM((1, GROUP), jnp.int32),
                pltpu.VMEM((16, LAT), jnp.float32),
                pltpu.VMEM_SHARED((qc + 8, LAT), jnp.float32),
            ]
        ),
    )
    def sk(vals_hbm, idx_hbm, out_hbm, *rest):
        vbufs = rest[:DEPTH]
        vsems = rest[DEPTH:2 * DEPTH]
        isems = rest[2 * DEPTH:2 * DEPTH + 2]
        ibufs = rest[2 * DEPTH + 2:2 * DEPTH + 4]
        adj_v, zbuf, acc = rest[2 * DEPTH + 4:]
        c = lax.axis_index("c")
        s = lax.axis_index("s")
        zbase = s * rows_per_tile
        ebase = s * (k2 * GROUP)

        def zrow(r, carry):
            for q in range(4):
                zbuf[r, pl.ds(q * 16, 16)] = jnp.zeros((16,), jnp.float32)
            return carry

        lax.fori_loop(0, 16, zrow, 0)

        for p in range(2):
            qbase = (2 * c + p) * qc

            def zcopy(z, carry):
                pltpu.sync_copy(zbuf, acc.at[pl.ds(zbase + z * 16, 16)])
                return carry

            lax.fori_loop(0, zblocks, zcopy, 0)
            plsc.subcore_barrier()

            for b in range(DEPTH):
                pltpu.async_copy(
                    vals_hbm.at[pl.ds(ebase + b * GROUP, GROUP)],
                    vbufs[b], vsems[b])
            pltpu.async_copy(idx_hbm.at[s, pl.ds(0, DEPTH)], ibufs[0],
                             isems[0])
            nsb = k2 // DEPTH

            def body(sbp, carry):
                for par in range(2):
                    sb = 2 * sbp + par
                    pltpu.make_async_copy(
                        idx_hbm.at[s, pl.ds(sb * DEPTH, DEPTH)],
                        ibufs[par], isems[par]).wait()

                    @pl.when(sb + 1 < nsb)
                    def _():
                        pltpu.async_copy(
                            idx_hbm.at[s, pl.ds((sb + 1) * DEPTH, DEPTH)],
                            ibufs[1 - par], isems[1 - par])
                    for b in range(DEPTH):
                        j = sb * DEPTH + b
                        pltpu.make_async_copy(
                            vals_hbm.at[pl.ds(ebase + j * GROUP, GROUP)],
                            vbufs[b], vsems[b]).wait()
                        for q in range(4):
                            pv = ibufs[par][b, pl.ds(q * 16, 16)]
                            for h in range(2):
                                v = (pv >> (16 * h)) & 0xFFFF
                                local = v - qbase
                                ok = (local >= 0) & (local < qc)
                                adj_v[0, pl.ds(q * 32 + 16 * h, 16)] = (
                                    jnp.where(ok, local, qc))
                        pltpu.sync_copy(vbufs[b], acc.at[adj_v.at[0]],
                                        add=True)

                        @pl.when(j + DEPTH < k2)
                        def _():
                            pltpu.async_copy(
                                vals_hbm.at[pl.ds(ebase + (j + DEPTH) * GROUP,
                                                  GROUP)],
                                vbufs[b], vsems[b])
                return carry

            lax.fori_loop(0, nsb // 2, body, 0)
            plsc.subcore_barrier()

            pltpu.sync_copy(
                acc.at[pl.ds(zbase, rows_per_tile)],
                out_hbm.at[pl.ds(qbase + zbase, rows_per_tile)],
            )

    return sk(vals, idx3)


def _propagate(node, edge_lat, send, recv3, msg_p, upd_p, hc):
    nj = _sc_gather(node, send)
    ne = _mlp2(nj, edge_lat, msg_p, residual=False)
    agg = _sc_segsum(ne, recv3, hc)
    return _mlp2(node, agg, upd_p, residual=True)


def kernel(sparse_x, sparse_edge_attr, dense_x, dense_edge_attr,
           multilayer_edge_attr, params, sparse_edge_index, dense_edge_index,
           multilayer_edge_index):
    es_pad = _pad_to(E_SPARSE)
    ed_pad = _pad_to(E_DENSE)
    em_pad = _pad_to(E_ML)

    def prep_idx(ei, epad, e):
        send = jnp.pad(ei[0].astype(jnp.int32), (0, epad - e))
        recv = jnp.pad(ei[1].astype(jnp.int32), (0, epad - e),
                       constant_values=DUMMY_RECV)
        return send, recv

    s_send, s_recv = prep_idx(sparse_edge_index, es_pad, E_SPARSE)
    d_send, d_recv = prep_idx(dense_edge_index, ed_pad, E_DENSE)
    m_send = jnp.pad(multilayer_edge_index[0].astype(jnp.int32),
                     (0, em_pad - E_ML))

    nls = _mlp1(_pad_rows(sparse_x, NPAD_SPARSE), params["enc_node_sparse"])
    els = _mlp1(_pad_rows(sparse_edge_attr, es_pad), params["enc_edge_sparse"])
    elm = _mlp1(_pad_rows(multilayer_edge_attr, em_pad), params["enc_edge_md"])
    nld = _mlp1(_pad_rows(dense_x, NPAD_DENSE), params["enc_node_dense"])
    eld = _mlp1(_pad_rows(dense_edge_attr, ed_pad), params["enc_edge_dense"])

    node = nls
    for i in range(PASSES):
        node = _propagate(node, els, s_send, s_recv,
                          params["sparse_msg"][i], params["sparse_upd"][i],
                          QC_SPARSE)
    upd_sparse = nls + node

    nj = _sc_gather(upd_sparse, m_send)
    ne = _mlp2(nj, elm, params["link_msg"], residual=False)
    agg = ne[:E_ML].reshape(N_DENSE, CLOSEST * LAT)
    upd_dense = _mlp1(_pad_rows(agg, NPAD_DENSE), params["link_upd"])
    node_d = nld + upd_dense

    node = node_d
    for i in range(PASSES):
        node = _propagate(node, eld, d_send, d_recv,
                          params["dense_msg"][i], params["dense_upd"][i],
                          QC_DENSE)

    final = node_d + node
    return _mlp1(final, params["decoder"])[:N_DENSE]


# speedup vs baseline: 1.1494x; 1.1494x over previous
import functools

import jax
import jax.numpy as jnp
from jax import lax
from jax.experimental import pallas as pl
from jax.experimental.pallas import tpu as pltpu
from jax.experimental.pallas import tpu_sc as plsc

N_SPARSE = 10000
N_DENSE = 50000
E_SPARSE = 160000
E_DENSE = 800000
CLOSEST = 3
E_ML = N_DENSE * CLOSEST
LAT = 64
PASSES = 8

NW = 32
GROUP = 128
SLOTS = 8
DEPTH = 4
EDGE_QUANT = NW * GROUP * DEPTH

QC_SPARSE = 2560
QC_DENSE = 12544
NPAD_SPARSE = 4 * QC_SPARSE
NPAD_DENSE = 4 * QC_DENSE

DUMMY_RECV = 0xFFFF


def _pad_rows(x, n):
    return jnp.pad(x, ((0, n - x.shape[0]), (0, 0)))


def _pad_to(e):
    return ((e + EDGE_QUANT - 1) // EDGE_QUANT) * EDGE_QUANT


def _mlp1_body(x_ref, w1_ref, b1_ref, w2_ref, b2_ref, o_ref):
    h = jnp.dot(x_ref[...], w1_ref[...], preferred_element_type=jnp.float32)
    h = jnp.maximum(h + b1_ref[...], 0.0)
    o_ref[...] = jnp.dot(h, w2_ref[...], preferred_element_type=jnp.float32) + b2_ref[...]


def _mlp1(x, p, block=2048):
    n, din = x.shape
    dout = p["W2"].shape[1]
    return pl.pallas_call(
        _mlp1_body,
        grid=(pl.cdiv(n, block),),
        in_specs=[
            pl.BlockSpec((block, din), lambda i: (i, 0)),
            pl.BlockSpec((din, LAT), lambda i: (0, 0)),
            pl.BlockSpec((1, LAT), lambda i: (0, 0)),
            pl.BlockSpec((LAT, dout), lambda i: (0, 0)),
            pl.BlockSpec((1, dout), lambda i: (0, 0)),
        ],
        out_specs=pl.BlockSpec((block, dout), lambda i: (i, 0)),
        out_shape=jax.ShapeDtypeStruct((n, dout), jnp.float32),
    )(x, p["W1"], p["b1"].reshape(1, -1), p["W2"], p["b2"].reshape(1, -1))


def _mlp2_body(a_ref, b_ref, w1a_ref, w1b_ref, b1_ref, w2_ref, b2_ref, o_ref,
               *, residual):
    h = jnp.dot(a_ref[...], w1a_ref[...], preferred_element_type=jnp.float32)
    h = h + jnp.dot(b_ref[...], w1b_ref[...], preferred_element_type=jnp.float32)
    h = jnp.maximum(h + b1_ref[...], 0.0)
    o = jnp.dot(h, w2_ref[...], preferred_element_type=jnp.float32) + b2_ref[...]
    if residual:
        o = o + a_ref[...]
    o_ref[...] = o


def _mlp2(a, b, p, residual, block=2048):
    n = a.shape[0]
    return pl.pallas_call(
        functools.partial(_mlp2_body, residual=residual),
        grid=(pl.cdiv(n, block),),
        in_specs=[
            pl.BlockSpec((block, LAT), lambda i: (i, 0)),
            pl.BlockSpec((block, LAT), lambda i: (i, 0)),
            pl.BlockSpec((LAT, LAT), lambda i: (0, 0)),
            pl.BlockSpec((LAT, LAT), lambda i: (0, 0)),
            pl.BlockSpec((1, LAT), lambda i: (0, 0)),
            pl.BlockSpec((LAT, LAT), lambda i: (0, 0)),
            pl.BlockSpec((1, LAT), lambda i: (0, 0)),
        ],
        out_specs=pl.BlockSpec((block, LAT), lambda i: (i, 0)),
        out_shape=jax.ShapeDtypeStruct((n, LAT), jnp.float32),
    )(a, b, p["W1"][:LAT], p["W1"][LAT:], p["b1"].reshape(1, -1),
      p["W2"], p["b2"].reshape(1, -1))


def _sc_gather(table, idx):
    epad = idx.shape[0]
    k = epad // (NW * GROUP)
    idx3 = idx.reshape(NW, k, GROUP)
    mesh = plsc.VectorSubcoreMesh(core_axis_name="c", subcore_axis_name="s")

    @functools.partial(
        pl.kernel,
        mesh=mesh,
        out_type=jax.ShapeDtypeStruct((epad, LAT), jnp.float32),
        compiler_params=pltpu.CompilerParams(use_tc_tiling_on_sc=False),
        scratch_types=(
            [pltpu.VMEM((k, GROUP), jnp.int32)]
            + [pltpu.VMEM((GROUP, LAT), jnp.float32) for _ in range(SLOTS)]
            + [pltpu.SemaphoreType.DMA for _ in range(2 * SLOTS)]
        ),
    )
    def gk(table_hbm, idx_hbm, out_hbm, idx_v, *rest):
        bufs = rest[:SLOTS]
        gsems = rest[SLOTS:2 * SLOTS]
        wsems = rest[2 * SLOTS:3 * SLOTS]
        wid = lax.axis_index("s") * 2 + lax.axis_index("c")
        pltpu.sync_copy(idx_hbm.at[wid], idx_v)
        base = wid * (k * GROUP)

        for b in range(DEPTH):
            pltpu.async_copy(table_hbm.at[idx_v.at[b]], bufs[b], gsems[b])

        def ring_iter(j, b):
            nb = (b + DEPTH) % SLOTS

            @pl.when(j >= DEPTH)
            def _():
                pltpu.make_async_copy(
                    bufs[nb],
                    out_hbm.at[pl.ds(base + (j - DEPTH) * GROUP, GROUP)],
                    wsems[nb]).wait()

            @pl.when(j + DEPTH < k)
            def _():
                pltpu.async_copy(table_hbm.at[idx_v.at[j + DEPTH]],
                                 bufs[nb], gsems[nb])
            pltpu.make_async_copy(table_hbm.at[idx_v.at[j]], bufs[b],
                                  gsems[b]).wait()
            pltpu.async_copy(
                bufs[b], out_hbm.at[pl.ds(base + j * GROUP, GROUP)],
                wsems[b])

        def body(sb, carry):
            for b in range(SLOTS):
                ring_iter(sb * SLOTS + b, b)
            return carry

        lax.fori_loop(0, k // SLOTS, body, 0)
        for i in range(k % SLOTS):
            ring_iter((k // SLOTS) * SLOTS + i, i)

        for i in range(DEPTH):
            j = k - DEPTH + i
            pltpu.make_async_copy(
                bufs[j % SLOTS],
                out_hbm.at[pl.ds(base + j * GROUP, GROUP)],
                wsems[j % SLOTS]).wait()

    return gk(table, idx3)


def _sc_segsum(vals, idx, qc):
    epad = vals.shape[0]
    k2 = epad // (16 * GROUP)
    r2 = idx.reshape(-1, 2, 16)
    packed = r2[:, 0, :] | (r2[:, 1, :] << 16)
    idx3 = packed.reshape(16, k2, 64)
    rows_per_tile = qc // 16
    zblocks = rows_per_tile // 16
    mesh = plsc.VectorSubcoreMesh(core_axis_name="c", subcore_axis_name="s")

    @functools.partial(
        pl.kernel,
        mesh=mesh,
        out_type=jax.ShapeDtypeStruct((4 * qc, LAT), jnp.float32),
        compiler_params=pltpu.CompilerParams(use_tc_tiling_on_sc=False),
        scratch_types=(
            [pltpu.VMEM((k2, 64), jnp.int32)]
            + [pltpu.VMEM((GROUP, LAT), jnp.float32) for _ in range(DEPTH)]
            + [pltpu.SemaphoreType.DMA for _ in range(DEPTH)]
            + [
                pltpu.VMEM((1, GROUP), jnp.int32),
                pltpu.VMEM((16, LAT), jnp.float32),
                pltpu.VMEM_SHARED((qc + 8, LAT), jnp.float32),
            ]
        ),
    )
    def sk(vals_hbm, idx_hbm, out_hbm, idx_v, *rest):
        vbufs = rest[:DEPTH]
        vsems = rest[DEPTH:2 * DEPTH]
        adj_v, zbuf, acc = rest[2 * DEPTH:]
        pltpu.sync_copy(idx_hbm.at[lax.axis_index("s")], idx_v)
        c = lax.axis_index("c")
        s = lax.axis_index("s")
        zbase = s * rows_per_tile
        ebase = s * (k2 * GROUP)

        def zrow(r, carry):
            for q in range(4):
                zbuf[r, pl.ds(q * 16, 16)] = jnp.zeros((16,), jnp.float32)
            return carry

        lax.fori_loop(0, 16, zrow, 0)

        for p in range(2):
            qbase = (2 * c + p) * qc

            def zcopy(z, carry):
                pltpu.sync_copy(zbuf, acc.at[pl.ds(zbase + z * 16, 16)])
                return carry

            lax.fori_loop(0, zblocks, zcopy, 0)
            plsc.subcore_barrier()

            for b in range(DEPTH):
                pltpu.async_copy(
                    vals_hbm.at[pl.ds(ebase + b * GROUP, GROUP)],
                    vbufs[b], vsems[b])

            def body(sb, carry):
                for b in range(DEPTH):
                    j = sb * DEPTH + b
                    pltpu.make_async_copy(
                        vals_hbm.at[pl.ds(ebase + j * GROUP, GROUP)],
                        vbufs[b], vsems[b]).wait()
                    for q in range(4):
                        pv = idx_v[j, pl.ds(q * 16, 16)]
                        for h in range(2):
                            v = (pv >> (16 * h)) & 0xFFFF
                            local = v - qbase
                            ok = (local >= 0) & (local < qc)
                            adj_v[0, pl.ds(q * 32 + 16 * h, 16)] = (
                                jnp.where(ok, local, qc))
                    pltpu.sync_copy(vbufs[b], acc.at[adj_v.at[0]], add=True)

                    @pl.when(j + DEPTH < k2)
                    def _():
                        pltpu.async_copy(
                            vals_hbm.at[pl.ds(ebase + (j + DEPTH) * GROUP,
                                              GROUP)],
                            vbufs[b], vsems[b])
                return carry

            lax.fori_loop(0, k2 // DEPTH, body, 0)
            plsc.subcore_barrier()

            pltpu.sync_copy(
                acc.at[pl.ds(zbase, rows_per_tile)],
                out_hbm.at[pl.ds(qbase + zbase, rows_per_tile)],
            )

    return sk(vals, idx3)


def _propagate(node, edge_lat, send, recv3, msg_p, upd_p, hc):
    nj = _sc_gather(node, send)
    ne = _mlp2(nj, edge_lat, msg_p, residual=False)
    agg = _sc_segsum(ne, recv3, hc)
    return _mlp2(node, agg, upd_p, residual=True)


def kernel(sparse_x, sparse_edge_attr, dense_x, dense_edge_attr,
           multilayer_edge_attr, params, sparse_edge_index, dense_edge_index,
           multilayer_edge_index):
    es_pad = _pad_to(E_SPARSE)
    ed_pad = _pad_to(E_DENSE)
    em_pad = _pad_to(E_ML)

    def prep_idx(ei, epad, e):
        send = jnp.pad(ei[0].astype(jnp.int32), (0, epad - e))
        recv = jnp.pad(ei[1].astype(jnp.int32), (0, epad - e),
                       constant_values=DUMMY_RECV)
        return send, recv

    s_send, s_recv = prep_idx(sparse_edge_index, es_pad, E_SPARSE)
    d_send, d_recv = prep_idx(dense_edge_index, ed_pad, E_DENSE)
    m_send = jnp.pad(multilayer_edge_index[0].astype(jnp.int32),
                     (0, em_pad - E_ML))

    nls = _mlp1(_pad_rows(sparse_x, NPAD_SPARSE), params["enc_node_sparse"])
    els = _mlp1(_pad_rows(sparse_edge_attr, es_pad), params["enc_edge_sparse"])
    elm = _mlp1(_pad_rows(multilayer_edge_attr, em_pad), params["enc_edge_md"])
    nld = _mlp1(_pad_rows(dense_x, NPAD_DENSE), params["enc_node_dense"])
    eld = _mlp1(_pad_rows(dense_edge_attr, ed_pad), params["enc_edge_dense"])

    node = nls
    for i in range(PASSES):
        node = _propagate(node, els, s_send, s_recv,
                          params["sparse_msg"][i], params["sparse_upd"][i],
                          QC_SPARSE)
    upd_sparse = nls + node

    nj = _sc_gather(upd_sparse, m_send)
    ne = _mlp2(nj, elm, params["link_msg"], residual=False)
    agg = ne[:E_ML].reshape(N_DENSE, CLOSEST * LAT)
    upd_dense = _mlp1(_pad_rows(agg, NPAD_DENSE), params["link_upd"])
    node_d = nld + upd_dense

    node = node_d
    for i in range(PASSES):
        node = _propagate(node, eld, d_send, d_recv,
                          params["dense_msg"][i], params["dense_upd"][i],
                          QC_DENSE)

    final = node_d + node
    return _mlp1(final, params["decoder"])[:N_DENSE]
